# Initial kernel scaffold; baseline (speedup 1.0000x reference)
import jax
import jax.numpy as jnp
from jax.experimental import pallas as pl


def kernel(logits, labels):
    raise NotImplementedError("write your pallas kernel here")


# speedup vs baseline: 1.0806x; 1.0806x over previous
import functools

import jax
import jax.numpy as jnp
from jax import lax
from jax.experimental import pallas as pl
from jax.experimental.pallas import tpu as pltpu

_S = 64.0
_M3 = 0.35

_B = 1024
_C = 100000
_BR = 8
_NB = _B // _BR


def _scale_mask_body(lab_ref, x_ref, o_ref):
    x = x_ref[...]
    lab = lab_ref[0, 0, :]
    cols = lax.broadcasted_iota(jnp.int32, x.shape, 1)
    mask = cols == lab[:, None]
    o_ref[...] = jnp.where(mask, (x - _M3) * _S, x * _S)


@jax.jit
def _scale_mask(logits, labels):
    labels3 = labels.reshape(_NB, 1, _BR)
    return pl.pallas_call(
        _scale_mask_body,
        grid=(_NB,),
        in_specs=[
            pl.BlockSpec((1, 1, _BR), lambda i: (i, 0, 0)),
            pl.BlockSpec((_BR, _C), lambda i: (i, 0)),
        ],
        out_specs=pl.BlockSpec((_BR, _C), lambda i: (i, 0)),
        out_shape=jax.ShapeDtypeStruct((_B, _C), jnp.float32),
    )(labels3, logits)


def kernel(logits, labels):
    return _scale_mask(logits, labels.astype(jnp.int32))
